# Initial kernel scaffold; baseline (speedup 1.0000x reference)
import jax
import jax.numpy as jnp
from jax.experimental import pallas as pl


def kernel(feature, pos, We0a, be0a, We0b, be0b, Wg1, bg1, Wg2, bg2, Wu1, bu1, Wue1a, bue1a, Wue1b, bue1b, Wu2, bu2, Wue2a, bue2a, Wue2b, bue2b, Wd1, bd1, Wd2, bd2, Wd3, bd3):
    raise NotImplementedError("write your pallas kernel here")


# speedup vs baseline: 14.9953x; 14.9953x over previous
import dataclasses
import functools

import jax
import jax.numpy as jnp
from jax import lax
from jax.experimental import pallas as pl
from jax.experimental.pallas import tpu as pltpu
from jax.experimental.pallas import tpu_sc as plsc

NB, NN = 2, 4096
ROWS = 256
NEG = -3.0e38
GW = 128

_DIMS = lambda: (((1,), (0,)), ((), ()))
_DIMS_T = lambda: (((1,), (1,)), ((), ()))


def _lrelu(x):
    return jnp.where(x >= 0, x, 0.2 * x)


def _mm(a, b):
    return lax.dot_general(a, b, _DIMS(), preferred_element_type=jnp.float32)


def _sq_lanes(x):
    ones = jnp.ones((8, x.shape[1]), jnp.float32)
    q = lax.dot_general(ones, x * x, _DIMS_T(),
                        preferred_element_type=jnp.float32,
                        precision=lax.Precision.HIGHEST)
    return q[0:1]


def _prep_body(x_ref, wu_ref, bu_ref, wv_ref, sq_ref, u_ref, v_ref):
    x = x_ref[0]
    sq_ref[0] = _sq_lanes(x)
    u_ref[0] = _mm(x, wu_ref[...]) + bu_ref[...]
    v_ref[...] = _mm(x, wv_ref[...])


def _prep(x, wu, bu, wv):
    n, c = x.shape[1], x.shape[2]
    o = wu.shape[1]
    full = lambda b: (0, 0)
    return pl.pallas_call(
        _prep_body,
        grid=(NB,),
        in_specs=[
            pl.BlockSpec((1, n, c), lambda b: (b, 0, 0)),
            pl.BlockSpec((c, o), full),
            pl.BlockSpec((1, o), full),
            pl.BlockSpec((c, o), full),
        ],
        out_specs=[
            pl.BlockSpec((1, 1, n), lambda b: (b, 0, 0)),
            pl.BlockSpec((1, n, o), lambda b: (b, 0, 0)),
            pl.BlockSpec((n, o), lambda b: (b, 0)),
        ],
        out_shape=[
            jax.ShapeDtypeStruct((NB, 1, n), jnp.float32),
            jax.ShapeDtypeStruct((NB, n, o), jnp.float32),
            jax.ShapeDtypeStruct((NB * n, o), jnp.float32),
        ],
        compiler_params=pltpu.CompilerParams(
            dimension_semantics=("parallel",)),
    )(x, wu, bu, wv)


def _dist_topk_body(k, kp, xr_ref, xa_ref, sq_ref, idx_ref):
    xr = xr_ref[0]
    xa = xa_ref[0]
    n = xa.shape[0]
    d = 2.0 * lax.dot_general(xr, xa, _DIMS_T(),
                              preferred_element_type=jnp.float32) - sq_ref[0]
    col = lax.broadcasted_iota(jnp.int32, (ROWS, n), 1)
    lane = lax.broadcasted_iota(jnp.int32, (ROWS, kp), 1)
    acc = jnp.zeros((ROWS, kp), jnp.int32)
    for t in range(k):
        m = jnp.max(d, axis=1, keepdims=True)
        cand = jnp.where(d == m, col, n)
        a = jnp.min(cand, axis=1, keepdims=True)
        acc = jnp.where(lane == t, a, acc)
        d = jnp.where(col == a, NEG, d)
    idx_ref[...] = acc + pl.program_id(0) * NN


def _dist_topk(x, sq, k, kp):
    n, c = x.shape[1], x.shape[2]
    return pl.pallas_call(
        functools.partial(_dist_topk_body, k, kp),
        grid=(NB, n // ROWS),
        in_specs=[
            pl.BlockSpec((1, ROWS, c), lambda b, i: (b, i, 0)),
            pl.BlockSpec((1, n, c), lambda b, i: (b, 0, 0)),
            pl.BlockSpec((1, 1, n), lambda b, i: (b, 0, 0)),
        ],
        out_specs=pl.BlockSpec((ROWS, kp), lambda b, i: (b * (n // ROWS) + i, 0)),
        out_shape=jax.ShapeDtypeStruct((NB * n, kp), jnp.int32),
        compiler_params=pltpu.CompilerParams(
            dimension_semantics=("parallel", "parallel")),
    )(x, x, sq)


def _sc_gather(table, idx):
    n = idx.shape[0]
    o = table.shape[1]
    idx2 = idx.reshape(1, n)
    mesh = plsc.VectorSubcoreMesh(core_axis_name="core",
                                  subcore_axis_name="subcore")
    cp = pltpu.CompilerParams()
    if "needs_layout_passes" in pltpu.CompilerParams.__dataclass_fields__:
        cp = dataclasses.replace(cp, needs_layout_passes=False)

    @functools.partial(
        pl.kernel,
        out_type=jax.ShapeDtypeStruct((n, o), table.dtype),
        mesh=mesh,
        compiler_params=cp,
    )
    def _gather_kernel(x_hbm, i_hbm, o_hbm):
        def body(i_vmem, o_vmem):
            pltpu.sync_copy(x_hbm.at[i_vmem.at[0]], o_vmem)

        pltpu.emit_pipeline(
            body,
            grid=(n // GW,),
            in_specs=[pl.BlockSpec((1, GW), index_map=lambda i: (0, i))],
            out_specs=[pl.BlockSpec((GW, o), index_map=lambda i: (i, 0))],
            core_axis_name=("core", "subcore"),
            dimension_semantics=(pltpu.PARALLEL,),
        )(i_hbm, o_hbm)

    return _gather_kernel(table, idx2)


def _edge_max(u, g_ref, w2, b2, k):
    acc = None
    for t in range(k):
        h = _lrelu(u + g_ref[t])
        y = _lrelu(_mm(h, w2) + b2)
        acc = y if acc is None else jnp.maximum(acc, y)
    return acc


def _ec1_body(u_ref, g_ref, w2_ref, b2_ref, wg1_ref, bg1_ref, wg2_ref,
              bg2_ref, wu1_ref, bu1_ref, p2u_ref, b2u_ref, p2v_ref,
              h64_ref, sq2_ref, u2_ref, v2_ref):
    x1 = _edge_max(u_ref[0], g_ref, w2_ref[...], b2_ref[...], 20)
    f1 = _lrelu(_mm(x1, wg1_ref[...]) + bg1_ref[...]) + x1
    f2 = _lrelu(_mm(f1, wg2_ref[...]) + bg2_ref[...]) + f1
    wu1 = wu1_ref[...]
    h64 = jnp.maximum(_mm(f1, wu1[:128]) + _mm(f2, wu1[128:]) + bu1_ref[...],
                      0.0)
    h64_ref[0] = h64
    sq2_ref[0] = _sq_lanes(h64)
    u2_ref[0] = _mm(h64, p2u_ref[...]) + b2u_ref[...]
    v2_ref[...] = _mm(h64, p2v_ref[...])


def _ec2_body(u_ref, g_ref, w2_ref, b2_ref, wu2_ref, bu2_ref, p3u_ref,
              b3u_ref, p3v_ref, h64_ref, sq3_ref, u3_ref, v3_ref):
    x2 = _edge_max(u_ref[0], g_ref, w2_ref[...], b2_ref[...], 12)
    h64 = jnp.maximum(_mm(x2, wu2_ref[...]) + bu2_ref[...], 0.0)
    h64_ref[0] = h64
    sq3_ref[0] = _sq_lanes(h64)
    u3_ref[0] = _mm(h64, p3u_ref[...]) + b3u_ref[...]
    v3_ref[...] = _mm(h64, p3v_ref[...])


def _ec3_body(u_ref, g_ref, w2_ref, b2_ref, wd1_ref, bd1_ref, wd2_ref,
              bd2_ref, wd3_ref, bd3_ref, pos_ref, newp_ref, edge_ref):
    x3 = _edge_max(u_ref[0], g_ref, w2_ref[...], b2_ref[...], 4)
    d1 = jnp.maximum(_mm(x3, wd1_ref[...]) + bd1_ref[...], 0.0)
    d2 = jnp.maximum(_mm(d1, wd2_ref[...]) + bd2_ref[...], 0.0)
    d3 = _mm(d2, wd3_ref[...]) + bd3_ref[...]
    edge_ref[0] = d3
    newp_ref[0] = pos_ref[0] + d3


def _full2(shape):
    return pl.BlockSpec(shape, lambda b, i: (0, 0))


def _edge_call(body, k, cin, cout, u, g, extras, out_specs, out_shapes):
    steps = NN // ROWS
    in_specs = [
        pl.BlockSpec((1, ROWS, cin), lambda b, i: (b, i, 0)),
        pl.BlockSpec((k, ROWS, cout), lambda b, i: (0, b * steps + i, 0)),
    ] + [_full2(e.shape) for e in extras]
    return pl.pallas_call(
        body,
        grid=(NB, steps),
        in_specs=in_specs,
        out_specs=out_specs,
        out_shape=out_shapes,
        compiler_params=pltpu.CompilerParams(
            dimension_semantics=("parallel", "parallel")),
    )(u, g, *extras)


def _flat_idx(idx, k):
    return idx[:, :k].T.reshape(-1)


def kernel(feature, pos, We0a, be0a, We0b, be0b, Wg1, bg1, Wg2, bg2, Wu1,
           bu1, Wue1a, bue1a, Wue1b, bue1b, Wu2, bu2, Wue2a, bue2a, Wue2b,
           bue2b, Wd1, bd1, Wd2, bd2, Wd3, bd3):
    f32 = jnp.float32
    r2 = lambda v: v.reshape(1, -1).astype(f32)
    steps = NN // ROWS

    p1u = (We0a[:, :128] - We0a[:, 128:]).T
    p1v = We0a[:, 128:].T
    sq1, u1, v1 = _prep(feature, p1u, r2(be0a), p1v)
    idx1 = _dist_topk(feature, sq1, 20, 24)
    g1 = _sc_gather(v1, _flat_idx(idx1, 20)).reshape(20, NB * NN, 128)

    p2u = (Wue1a[:, :64] - Wue1a[:, 64:]).T
    p2v = Wue1a[:, 64:].T
    h64a, sq2, u2, v2 = _edge_call(
        _ec1_body, 20, 128, 128, u1, g1,
        [We0b.T, r2(be0b), Wg1.T, r2(bg1), Wg2.T, r2(bg2), Wu1.T, r2(bu1),
         p2u, r2(bue1a), p2v],
        [
            pl.BlockSpec((1, ROWS, 64), lambda b, i: (b, i, 0)),
            pl.BlockSpec((1, 1, ROWS), lambda b, i: (b, 0, i)),
            pl.BlockSpec((1, ROWS, 256), lambda b, i: (b, i, 0)),
            pl.BlockSpec((ROWS, 256), lambda b, i: (b * steps + i, 0)),
        ],
        [
            jax.ShapeDtypeStruct((NB, NN, 64), f32),
            jax.ShapeDtypeStruct((NB, 1, NN), f32),
            jax.ShapeDtypeStruct((NB, NN, 256), f32),
            jax.ShapeDtypeStruct((NB * NN, 256), f32),
        ])

    idx2 = _dist_topk(h64a, sq2, 12, 16)
    g2 = _sc_gather(v2, _flat_idx(idx2, 12)).reshape(12, NB * NN, 256)

    p3u = (Wue2a[:, :64] - Wue2a[:, 64:]).T
    p3v = Wue2a[:, 64:].T
    h64b, sq3, u3, v3 = _edge_call(
        _ec2_body, 12, 256, 256, u2, g2,
        [Wue1b.T, r2(bue1b), Wu2.T, r2(bu2), p3u, r2(bue2a), p3v],
        [
            pl.BlockSpec((1, ROWS, 64), lambda b, i: (b, i, 0)),
            pl.BlockSpec((1, 1, ROWS), lambda b, i: (b, 0, i)),
            pl.BlockSpec((1, ROWS, 256), lambda b, i: (b, i, 0)),
            pl.BlockSpec((ROWS, 256), lambda b, i: (b * steps + i, 0)),
        ],
        [
            jax.ShapeDtypeStruct((NB, NN, 64), f32),
            jax.ShapeDtypeStruct((NB, 1, NN), f32),
            jax.ShapeDtypeStruct((NB, NN, 256), f32),
            jax.ShapeDtypeStruct((NB * NN, 256), f32),
        ])

    idx3 = _dist_topk(h64b, sq3, 4, 8)
    g3 = _sc_gather(v3, _flat_idx(idx3, 4)).reshape(4, NB * NN, 256)

    pos24 = jnp.concatenate([pos] * 8, axis=-1)
    in_specs = [
        pl.BlockSpec((1, ROWS, 256), lambda b, i: (b, i, 0)),
        pl.BlockSpec((4, ROWS, 256), lambda b, i: (0, b * steps + i, 0)),
        _full2(Wue2b.T.shape), _full2((1, 256)),
        _full2(Wd1.T.shape), _full2((1, 12)),
        _full2(Wd2.T.shape), _full2((1, 24)),
        _full2(Wd3.T.shape), _full2((1, 24)),
        pl.BlockSpec((1, ROWS, 24), lambda b, i: (b, i, 0)),
    ]
    newp, edge = pl.pallas_call(
        _ec3_body,
        grid=(NB, steps),
        in_specs=in_specs,
        out_specs=[
            pl.BlockSpec((1, ROWS, 24), lambda b, i: (b, i, 0)),
            pl.BlockSpec((1, ROWS, 24), lambda b, i: (b, i, 0)),
        ],
        out_shape=[
            jax.ShapeDtypeStruct((NB, NN, 24), f32),
            jax.ShapeDtypeStruct((NB, NN, 24), f32),
        ],
        compiler_params=pltpu.CompilerParams(
            dimension_semantics=("parallel", "parallel")),
    )(u3, g3, Wue2b.T, r2(bue2b), Wd1.T, r2(bd1), Wd2.T, r2(bd2), Wd3.T,
      r2(bd3), pos24)

    return (newp.reshape(NB, NN * 8, 3), edge.reshape(NB, NN * 8, 3))
